# baseline (speedup 1.0000x reference)
import jax
import jax.numpy as jnp
from jax import lax
from jax.experimental import pallas as pl
from jax.experimental.pallas import tpu as pltpu

N = 8192
D = 2048
K = 1024
RB = 1024
NBLK = N // RB
F32 = jnp.float32
BF16 = jnp.bfloat16


def _fused_body(lab_ref, pred_ref, f_ref, acc_ref, csum_ref, cntc_ref,
                cntr_ref, pb_ref, xxc_ref, xxr_ref):
    i = pl.program_id(0)
    lane = lax.broadcasted_iota(jnp.int32, (1, 128), 1)

    @pl.when(i == 0)
    def _():
        acc_ref[...] = jnp.where(lane == 2, jnp.inf,
                                 jnp.where(lane == 3, -jnp.inf, 0.0))
        csum_ref[...] = jnp.zeros_like(csum_ref)
        cntc_ref[...] = jnp.zeros_like(cntc_ref)
        cntr_ref[...] = jnp.zeros_like(cntr_ref)

    @pl.when(i < NBLK)
    def _():
        lab_f = lab_ref[...]
        lab = lab_f.astype(jnp.int32)
        ohT = (lax.broadcasted_iota(jnp.int32, (K, RB), 0) == lab)
        ohTb = ohT.astype(BF16)
        csum_ref[...] += lax.dot_general(
            ohTb, f_ref[...].astype(BF16), (((1,), (0,)), ((), ())),
            preferred_element_type=F32)
        ohTf = ohT.astype(F32)
        cntc_ref[...] += jnp.sum(ohTf, axis=1, keepdims=True)
        ones = jnp.ones((1, RB), F32)
        cntr_ref[...] += lax.dot_general(
            ones, ohTf, (((1,), (1,)), ((), ())),
            preferred_element_type=F32)
        e = lab_f - pred_ref[...]
        acc_ref[...] += jnp.where(lane == 7, jnp.sum(e * e), 0.0)

    @pl.when(i == NBLK)
    def _():
        cnt = cntc_ref[...]
        c = csum_ref[...] / jnp.where(cnt > 0, cnt, 1.0)
        nrm = jnp.maximum(jnp.sqrt(jnp.sum(c * c, axis=1, keepdims=True)),
                          1e-12)
        p = c / nrm
        xxc = jnp.sum(p * p, axis=1, keepdims=True)
        psq = p * p
        ones = jnp.ones((1, D), F32)
        xxr = lax.dot_general(ones, psq, (((1,), (1,)), ((), ())),
                              preferred_element_type=F32)
        pb = p.astype(BF16)
        pb_ref[...] = pb
        xxc_ref[...] = xxc
        xxr_ref[...] = xxr
        npres = jnp.sum((cntr_ref[...] > 0).astype(F32))
        acc_ref[...] += jnp.where(lane == 4, npres, 0.0)

    @pl.when(i > NBLK)
    def _():
        lab = lab_ref[...].astype(jnp.int32)
        ohTb = (lax.broadcasted_iota(jnp.int32, (K, RB), 0) == lab).astype(BF16)
        fc = lax.dot_general(ohTb, pb_ref[...],
                             (((0,), (0,)), ((), ())),
                             preferred_element_type=F32)
        dif = f_ref[...] - fc
        t = jnp.sum(dif * dif, axis=1, keepdims=True)
        mask = t > 0
        s = jnp.sqrt(jnp.where(mask, t, 1.0))
        ssum = jnp.sum(jnp.where(mask, s, 0.0))
        scnt = jnp.sum(mask.astype(F32))

        SW = K // NBLK
        j = i - NBLK - 1
        pbj = pb_ref[pl.ds(j * SW, SW), :]
        g = lax.dot_general(pb_ref[...], pbj, (((1,), (1,)), ((), ())),
                            preferred_element_type=F32)
        d = xxc_ref[...] + xxr_ref[:, pl.ds(j * SW, SW)] - 2.0 * g
        dist = jnp.sqrt(jnp.maximum(d, 1e-12))
        gi = lax.broadcasted_iota(jnp.int32, (K, SW), 0)
        gj = lax.broadcasted_iota(jnp.int32, (K, SW), 1) + j * SW
        wm = jnp.abs(gi - gj).astype(F32)
        pmask = ((gj > gi) & (cntc_ref[...] > 0)
                 & (cntr_ref[:, pl.ds(j * SW, SW)] > 0))
        s0 = jnp.sum(jnp.where(pmask, dist, 0.0))
        s1 = jnp.sum(jnp.where(pmask, dist * wm, 0.0))
        wmn = jnp.min(jnp.where(pmask, wm, jnp.inf))
        wmx = jnp.max(jnp.where(pmask, wm, -jnp.inf))

        r = acc_ref[...]
        r = (r + jnp.where(lane == 0, s0, 0.0)
             + jnp.where(lane == 1, s1, 0.0)
             + jnp.where(lane == 5, ssum, 0.0)
             + jnp.where(lane == 6, scnt, 0.0))
        r = jnp.where(lane == 2, jnp.minimum(r, wmn), r)
        r = jnp.where(lane == 3, jnp.maximum(r, wmx), r)
        acc_ref[...] = r


def kernel(features, labels, preds):
    lab3 = labels.reshape(NBLK, 1, RB)
    pred3 = preds.reshape(NBLK, 1, RB)

    def fmap(i):
        return (jnp.where(i < NBLK, i, jnp.maximum(i - (NBLK + 1), 0)), 0)

    def lmap(i):
        return (jnp.where(i < NBLK, i, jnp.maximum(i - (NBLK + 1), 0)), 0, 0)

    acc = pl.pallas_call(
        _fused_body,
        grid=(2 * NBLK + 1,),
        in_specs=[
            pl.BlockSpec((None, 1, RB), lmap),
            pl.BlockSpec((None, 1, RB), lmap),
            pl.BlockSpec((RB, D), fmap),
        ],
        out_specs=pl.BlockSpec((1, 128), lambda i: (0, 0)),
        out_shape=jax.ShapeDtypeStruct((1, 128), F32),
        scratch_shapes=[
            pltpu.VMEM((K, D), F32),
            pltpu.VMEM((K, 1), F32),
            pltpu.VMEM((1, K), F32),
            pltpu.VMEM((K, D), BF16),
            pltpu.VMEM((K, 1), F32),
            pltpu.VMEM((1, K), F32),
        ],
    )(lab3, pred3, features)

    s0 = acc[0, 0]
    s1 = acc[0, 1]
    wmn = acc[0, 2]
    wmx = acc[0, 3]
    npres = acc[0, 4]
    ssum = acc[0, 5]
    scnt = acc[0, 6]
    sse = acc[0, 7]

    n_pairs = npres * (npres - 1.0) * 0.5
    entropy = (s1 - wmn * s0) / wmx / n_pairs
    tight = ssum / jnp.maximum(scnt, 1.0)
    mse = sse / N
    return mse + 0.001 * (tight - entropy)


# speedup vs baseline: 1.0422x; 1.0422x over previous
import jax
import jax.numpy as jnp
from jax import lax
from jax.experimental import pallas as pl
from jax.experimental.pallas import tpu as pltpu

N = 8192
D = 2048
K = 1024
RB = 1024
NBLK = N // RB
F32 = jnp.float32
BF16 = jnp.bfloat16


def _fused_body(lab_ref, pred_ref, f_ref, acc_ref, csum_ref, cntc_ref,
                cntr_ref, pb_ref):
    i = pl.program_id(0)
    lane = lax.broadcasted_iota(jnp.int32, (1, 128), 1)

    @pl.when(i == 0)
    def _():
        acc_ref[...] = jnp.where(lane == 2, jnp.inf,
                                 jnp.where(lane == 3, -jnp.inf, 0.0))
        csum_ref[...] = jnp.zeros_like(csum_ref)
        cntc_ref[...] = jnp.zeros_like(cntc_ref)
        cntr_ref[...] = jnp.zeros_like(cntr_ref)

    @pl.when(i < NBLK)
    def _():
        lab_f = lab_ref[...]
        lab = lab_f.astype(jnp.int32)
        ohT = (lax.broadcasted_iota(jnp.int32, (K, RB), 0) == lab)
        ohTb = ohT.astype(BF16)
        csum_ref[...] += lax.dot_general(
            ohTb, f_ref[...].astype(BF16), (((1,), (0,)), ((), ())),
            preferred_element_type=F32)
        ohTf = ohT.astype(F32)
        cntc_ref[...] += jnp.sum(ohTf, axis=1, keepdims=True)
        ones = jnp.ones((1, RB), F32)
        cntr_ref[...] += lax.dot_general(
            ones, ohTf, (((1,), (1,)), ((), ())),
            preferred_element_type=F32)
        e = lab_f - pred_ref[...]
        acc_ref[...] += jnp.where(lane == 7, jnp.sum(e * e), 0.0)

    @pl.when(i == NBLK)
    def _():
        cnt = cntc_ref[...]
        c = csum_ref[...] / jnp.where(cnt > 0, cnt, 1.0)
        nrm = jnp.maximum(jnp.sqrt(jnp.sum(c * c, axis=1, keepdims=True)),
                          1e-12)
        p = c / nrm
        xxc = jnp.sum(p * p, axis=1, keepdims=True)
        psq = p * p
        ones = jnp.ones((1, D), F32)
        xxr = lax.dot_general(ones, psq, (((1,), (1,)), ((), ())),
                              preferred_element_type=F32)
        pb = p.astype(BF16)
        pb_ref[...] = pb
        g = lax.dot_general(pb, pb, (((1,), (1,)), ((), ())),
                            preferred_element_type=F32)
        d = xxc + xxr - 2.0 * g
        dist = jnp.sqrt(jnp.maximum(d, 1e-12))
        gi = lax.broadcasted_iota(jnp.int32, (K, K), 0)
        gj = lax.broadcasted_iota(jnp.int32, (K, K), 1)
        wm = jnp.abs(gi - gj).astype(F32)
        mask = (gj > gi) & (cntc_ref[...] > 0) & (cntr_ref[...] > 0)
        s0 = jnp.sum(jnp.where(mask, dist, 0.0))
        s1 = jnp.sum(jnp.where(mask, dist * wm, 0.0))
        wmn = jnp.min(jnp.where(mask, wm, jnp.inf))
        wmx = jnp.max(jnp.where(mask, wm, -jnp.inf))
        npres = jnp.sum((cntr_ref[...] > 0).astype(F32))
        r = acc_ref[...]
        r = (r + jnp.where(lane == 0, s0, 0.0)
             + jnp.where(lane == 1, s1, 0.0)
             + jnp.where(lane == 4, npres, 0.0))
        r = jnp.where(lane == 2, jnp.minimum(r, wmn), r)
        r = jnp.where(lane == 3, jnp.maximum(r, wmx), r)
        acc_ref[...] = r

    @pl.when(i > NBLK)
    def _():
        lab = lab_ref[...].astype(jnp.int32)
        ohTb = (lax.broadcasted_iota(jnp.int32, (K, RB), 0) == lab).astype(BF16)
        fc = lax.dot_general(ohTb, pb_ref[...],
                             (((0,), (0,)), ((), ())),
                             preferred_element_type=F32)
        dif = f_ref[...] - fc
        t = jnp.sum(dif * dif, axis=1, keepdims=True)
        mask = t > 0
        s = jnp.sqrt(jnp.where(mask, t, 1.0))
        ssum = jnp.sum(jnp.where(mask, s, 0.0))
        scnt = jnp.sum(mask.astype(F32))
        acc_ref[...] += (jnp.where(lane == 5, ssum, 0.0)
                         + jnp.where(lane == 6, scnt, 0.0))


def kernel(features, labels, preds):
    lab3 = labels.reshape(NBLK, 1, RB)
    pred3 = preds.reshape(NBLK, 1, RB)

    def fmap(i):
        return (jnp.where(i < NBLK, i, jnp.maximum(i - (NBLK + 1), 0)), 0)

    def lmap(i):
        return (jnp.where(i < NBLK, i, jnp.maximum(i - (NBLK + 1), 0)), 0, 0)

    acc = pl.pallas_call(
        _fused_body,
        grid=(2 * NBLK + 1,),
        in_specs=[
            pl.BlockSpec((None, 1, RB), lmap),
            pl.BlockSpec((None, 1, RB), lmap),
            pl.BlockSpec((RB, D), fmap),
        ],
        out_specs=pl.BlockSpec((1, 128), lambda i: (0, 0)),
        out_shape=jax.ShapeDtypeStruct((1, 128), F32),
        scratch_shapes=[
            pltpu.VMEM((K, D), F32),
            pltpu.VMEM((K, 1), F32),
            pltpu.VMEM((1, K), F32),
            pltpu.VMEM((K, D), BF16),
        ],
    )(lab3, pred3, features)

    s0 = acc[0, 0]
    s1 = acc[0, 1]
    wmn = acc[0, 2]
    wmx = acc[0, 3]
    npres = acc[0, 4]
    ssum = acc[0, 5]
    scnt = acc[0, 6]
    sse = acc[0, 7]

    n_pairs = npres * (npres - 1.0) * 0.5
    entropy = (s1 - wmn * s0) / wmx / n_pairs
    tight = ssum / jnp.maximum(scnt, 1.0)
    mse = sse / N
    return mse + 0.001 * (tight - entropy)
